# baseline (speedup 1.0000x reference)
import jax
import jax.numpy as jnp
from jax.experimental import pallas as pl
from jax.experimental.pallas import tpu as pltpu

N = 2048
D = 1024
H = 8
HD = D // H
E = 8
I = 2 * HD
TB = 256


def _mhmoe_block(x_ref, w_hpt_ref, b_hp_ref, embt_ref, w_up_ref,
                 w_down_ref, w_opt_ref, b_op_ref, y_ref):
    x = x_ref[...].astype(jnp.bfloat16)
    h = jnp.dot(x, w_hpt_ref[...], preferred_element_type=jnp.float32)
    h = h + b_hp_ref[...]
    hb = h.astype(jnp.bfloat16)
    out = None
    for hh in range(H):
        hs = h[:, hh * HD:(hh + 1) * HD]
        hsb = hb[:, hh * HD:(hh + 1) * HD]
        logits = jnp.dot(hs, embt_ref[...],
                         preferred_element_type=jnp.float32)
        m = jnp.max(logits, axis=1, keepdims=True)
        ex = jnp.exp(logits - m)
        v = ex / jnp.sum(ex, axis=1, keepdims=True)
        iota = jax.lax.broadcasted_iota(jnp.int32, (TB, E), 1)
        m1 = jnp.max(v, axis=1, keepdims=True)
        i1 = jnp.min(jnp.where(v == m1, iota, E), axis=1, keepdims=True)
        sel1 = iota == i1
        vm = jnp.where(sel1, -1.0, v)
        m2 = jnp.max(vm, axis=1, keepdims=True)
        i2 = jnp.min(jnp.where(vm == m2, iota, E), axis=1, keepdims=True)
        sel2 = iota == i2
        w = jnp.where(sel1, m1, 0.0) + jnp.where(sel2, m2, 0.0)

        up = jnp.dot(hsb, w_up_ref[...],
                     preferred_element_type=jnp.float32)
        a = jnp.square(jnp.maximum(up, 0.0))
        a = (a.reshape(TB, E, I) * w[:, :, None]).reshape(TB, E * I)
        dn = jnp.dot(a.astype(jnp.bfloat16), w_down_ref[...],
                     preferred_element_type=jnp.float32)
        contrib = jnp.dot(dn.astype(jnp.bfloat16),
                          w_opt_ref[hh * HD:(hh + 1) * HD, :],
                          preferred_element_type=jnp.float32)
        out = contrib if out is None else out + contrib
    y_ref[...] = out + b_op_ref[...]


@jax.jit
def kernel(x, W_hp, b_hp, expert_emb, W_up, W_down, W_op, b_op):
    W_hpT = W_hp.T.astype(jnp.bfloat16)
    embT = expert_emb.T
    W_up_r = W_up.transpose(1, 0, 2).reshape(HD, E * I).astype(jnp.bfloat16)
    W_down_r = W_down.reshape(E * I, HD).astype(jnp.bfloat16)
    W_opT = W_op.T.astype(jnp.bfloat16)
    b_hp2 = b_hp.reshape(1, D)
    b_op2 = b_op.reshape(1, D)
    return pl.pallas_call(
        _mhmoe_block,
        grid=(N // TB,),
        in_specs=[
            pl.BlockSpec((TB, D), lambda i: (i, 0)),
            pl.BlockSpec((D, D), lambda i: (0, 0)),
            pl.BlockSpec((1, D), lambda i: (0, 0)),
            pl.BlockSpec((HD, E), lambda i: (0, 0)),
            pl.BlockSpec((HD, E * I), lambda i: (0, 0)),
            pl.BlockSpec((E * I, HD), lambda i: (0, 0)),
            pl.BlockSpec((D, D), lambda i: (0, 0)),
            pl.BlockSpec((1, D), lambda i: (0, 0)),
        ],
        out_specs=pl.BlockSpec((TB, D), lambda i: (i, 0)),
        out_shape=jax.ShapeDtypeStruct((N, D), jnp.float32),
        compiler_params=pltpu.CompilerParams(
            dimension_semantics=("parallel",)),
    )(x, W_hpT, b_hp2, embT, W_up_r, W_down_r, W_opT, b_op2)


# speedup vs baseline: 1.4255x; 1.4255x over previous
import jax
import jax.numpy as jnp
from jax.experimental import pallas as pl
from jax.experimental.pallas import tpu as pltpu

N = 2048
D = 1024
H = 8
HD = D // H
E = 8
I = 2 * HD
TB = 256


def _mhmoe_block(x_ref, w_hpt_ref, b_hp_ref, embt_ref, w_up_ref,
                 w_down_ref, w_opt_ref, b_op_ref, y_ref):
    x = x_ref[...].astype(jnp.bfloat16)
    h = jnp.dot(x, w_hpt_ref[...], preferred_element_type=jnp.float32)
    h = h + b_hp_ref[...]
    hb = h.astype(jnp.bfloat16)
    dns = []
    for hh in range(H):
        hs = h[:, hh * HD:(hh + 1) * HD]
        hsb = hb[:, hh * HD:(hh + 1) * HD]
        logits = jnp.dot(hs, embt_ref[...],
                         preferred_element_type=jnp.float32)
        m = jnp.max(logits, axis=1, keepdims=True)
        ex = jnp.exp(logits - m)
        v = ex / jnp.sum(ex, axis=1, keepdims=True)
        iota = jax.lax.broadcasted_iota(jnp.int32, (TB, E), 1)
        m1 = jnp.max(v, axis=1, keepdims=True)
        i1 = jnp.min(jnp.where(v == m1, iota, E), axis=1, keepdims=True)
        sel1 = iota == i1
        vm = jnp.where(sel1, -1.0, v)
        m2 = jnp.max(vm, axis=1, keepdims=True)
        i2 = jnp.min(jnp.where(vm == m2, iota, E), axis=1, keepdims=True)
        sel2 = iota == i2
        w = jnp.where(sel1, m1, 0.0) + jnp.where(sel2, m2, 0.0)

        up = jnp.dot(hsb, w_up_ref[...],
                     preferred_element_type=jnp.float32)
        parts = []
        for e in range(E):
            ue = up[:, e * I:(e + 1) * I]
            ae = jnp.square(jnp.maximum(ue, 0.0)) * w[:, e:e + 1]
            parts.append(ae.astype(jnp.bfloat16))
        a = jnp.concatenate(parts, axis=1)
        dn = jnp.dot(a, w_down_ref[...],
                     preferred_element_type=jnp.float32)
        dns.append(dn.astype(jnp.bfloat16))
    dn_all = jnp.concatenate(dns, axis=1)
    y = jnp.dot(dn_all, w_opt_ref[...], preferred_element_type=jnp.float32)
    y_ref[...] = y + b_op_ref[...]


@jax.jit
def kernel(x, W_hp, b_hp, expert_emb, W_up, W_down, W_op, b_op):
    W_hpT = W_hp.T.astype(jnp.bfloat16)
    embT = expert_emb.T
    W_up_r = W_up.transpose(1, 0, 2).reshape(HD, E * I).astype(jnp.bfloat16)
    W_down_r = W_down.reshape(E * I, HD).astype(jnp.bfloat16)
    W_opT = W_op.T.astype(jnp.bfloat16)
    b_hp2 = b_hp.reshape(1, D)
    b_op2 = b_op.reshape(1, D)
    return pl.pallas_call(
        _mhmoe_block,
        grid=(N // TB,),
        in_specs=[
            pl.BlockSpec((TB, D), lambda i: (i, 0)),
            pl.BlockSpec((D, D), lambda i: (0, 0)),
            pl.BlockSpec((1, D), lambda i: (0, 0)),
            pl.BlockSpec((HD, E), lambda i: (0, 0)),
            pl.BlockSpec((HD, E * I), lambda i: (0, 0)),
            pl.BlockSpec((E * I, HD), lambda i: (0, 0)),
            pl.BlockSpec((D, D), lambda i: (0, 0)),
            pl.BlockSpec((1, D), lambda i: (0, 0)),
        ],
        out_specs=pl.BlockSpec((TB, D), lambda i: (i, 0)),
        out_shape=jax.ShapeDtypeStruct((N, D), jnp.float32),
        compiler_params=pltpu.CompilerParams(
            dimension_semantics=("parallel",)),
    )(x, W_hpT, b_hp2, embT, W_up_r, W_down_r, W_opT, b_op2)


# speedup vs baseline: 2.0771x; 1.4571x over previous
import jax
import jax.numpy as jnp
from jax.experimental import pallas as pl
from jax.experimental.pallas import tpu as pltpu

N = 2048
D = 1024
H = 8
HD = D // H
E = 8
I = 2 * HD
TB = 512


def _mhmoe_block(x_ref, w_hpt_ref, b_hp_ref, embt_ref, w_up_ref,
                 w_down_ref, w_opt_ref, b_op_ref, y_ref):
    x = x_ref[...].astype(jnp.bfloat16)
    h = jnp.dot(x, w_hpt_ref[...], preferred_element_type=jnp.float32)
    h = h + b_hp_ref[...]
    hb = h.astype(jnp.bfloat16)
    dns = []
    for hh in range(H):
        hs = h[:, hh * HD:(hh + 1) * HD]
        hsb = hb[:, hh * HD:(hh + 1) * HD]
        logits = jnp.dot(hs, embt_ref[...],
                         preferred_element_type=jnp.float32)
        m = jnp.max(logits, axis=1, keepdims=True)
        ex = jnp.exp(logits - m)
        v = ex / jnp.sum(ex, axis=1, keepdims=True)
        iota = jax.lax.broadcasted_iota(jnp.int32, (TB, E), 1)
        m1 = jnp.max(v, axis=1, keepdims=True)
        i1 = jnp.min(jnp.where(v == m1, iota, E), axis=1, keepdims=True)
        sel1 = iota == i1
        vm = jnp.where(sel1, -1.0, v)
        m2 = jnp.max(vm, axis=1, keepdims=True)
        i2 = jnp.min(jnp.where(vm == m2, iota, E), axis=1, keepdims=True)
        sel2 = iota == i2
        w = jnp.where(sel1, m1, 0.0) + jnp.where(sel2, m2, 0.0)

        up = jnp.dot(hsb, w_up_ref[...],
                     preferred_element_type=jnp.float32
                     ).astype(jnp.bfloat16)
        wb = w.astype(jnp.bfloat16)
        parts = []
        for e in range(E):
            ue = up[:, e * I:(e + 1) * I]
            ae = jnp.square(jnp.maximum(ue, 0.0)) * wb[:, e:e + 1]
            parts.append(ae)
        a = jnp.concatenate(parts, axis=1)
        dn = jnp.dot(a, w_down_ref[...],
                     preferred_element_type=jnp.float32)
        dns.append(dn.astype(jnp.bfloat16))
    dn_all = jnp.concatenate(dns, axis=1)
    y = jnp.dot(dn_all, w_opt_ref[...], preferred_element_type=jnp.float32)
    y_ref[...] = y + b_op_ref[...]


@jax.jit
def kernel(x, W_hp, b_hp, expert_emb, W_up, W_down, W_op, b_op):
    W_hpT = W_hp.T.astype(jnp.bfloat16)
    embT = expert_emb.T
    W_up_r = W_up.transpose(1, 0, 2).reshape(HD, E * I).astype(jnp.bfloat16)
    W_down_r = W_down.reshape(E * I, HD).astype(jnp.bfloat16)
    W_opT = W_op.T.astype(jnp.bfloat16)
    b_hp2 = b_hp.reshape(1, D)
    b_op2 = b_op.reshape(1, D)
    return pl.pallas_call(
        _mhmoe_block,
        grid=(N // TB,),
        in_specs=[
            pl.BlockSpec((TB, D), lambda i: (i, 0)),
            pl.BlockSpec((D, D), lambda i: (0, 0)),
            pl.BlockSpec((1, D), lambda i: (0, 0)),
            pl.BlockSpec((HD, E), lambda i: (0, 0)),
            pl.BlockSpec((HD, E * I), lambda i: (0, 0)),
            pl.BlockSpec((E * I, HD), lambda i: (0, 0)),
            pl.BlockSpec((D, D), lambda i: (0, 0)),
            pl.BlockSpec((1, D), lambda i: (0, 0)),
        ],
        out_specs=pl.BlockSpec((TB, D), lambda i: (i, 0)),
        out_shape=jax.ShapeDtypeStruct((N, D), jnp.float32),
        compiler_params=pltpu.CompilerParams(
            dimension_semantics=("parallel",)),
    )(x, W_hpT, b_hp2, embT, W_up_r, W_down_r, W_opT, b_op2)
